# baseline (speedup 1.0000x reference)
import functools

import jax
import jax.numpy as jnp
from jax import lax
from jax.experimental import pallas as pl
from jax.experimental.pallas import tpu as pltpu
from jax.experimental.pallas import tpu_sc as plsc

NC = 2
NS = 16
NW = NC * NS
CHUNK = 80
BLK = 2000


@functools.lru_cache()
def _deg_kernel(n, e):
    epw = e // NW
    nchunks = epw // CHUNK
    mesh = plsc.VectorSubcoreMesh(core_axis_name="c", subcore_axis_name="s",
                                  num_cores=NC, num_subcores=NS)

    depth = 8

    @functools.partial(
        pl.kernel,
        out_type=jax.ShapeDtypeStruct((NC * n,), jnp.float32),
        mesh=mesh,
        scratch_types=[
            pltpu.VMEM((CHUNK,), jnp.float32),
            pltpu.VMEM((nchunks, CHUNK), jnp.int32),
            pltpu.VMEM((n,), jnp.float32),
            pltpu.VMEM_SHARED((n,), jnp.float32),
            pltpu.SemaphoreType.DMA,
        ],
    )
    def deg(dst_hbm, out_hbm, ones_v, didx, zbuf, deg_sh, sem):
        c = lax.axis_index("c")
        s = lax.axis_index("s")
        w = c * NS + s

        for j in range(CHUNK // 16):
            ones_v[pl.ds(j * 16, 16)] = jnp.ones((16,), jnp.float32)
        pltpu.sync_copy(dst_hbm.at[w], didx)

        @pl.when(s == 0)
        def _():
            def zb(i, carry):
                zbuf[pl.ds(i * 16, 16)] = jnp.zeros((16,), jnp.float32)
                return carry

            lax.fori_loop(0, n // 16, zb, 0)
            pltpu.sync_copy(zbuf, deg_sh)

        plsc.subcore_barrier()

        def fire(k):
            pltpu.async_copy(ones_v, deg_sh.at[didx.at[k]], sem, add=True)

        def drain():
            pltpu.make_async_copy(ones_v, deg_sh.at[didx.at[0]], sem).wait()

        for k in range(depth):
            fire(k)

        def body(k, carry):
            drain()
            fire(k + depth)
            return carry

        lax.fori_loop(0, nchunks - depth, body, 0)
        for _ in range(depth):
            drain()
        plsc.subcore_barrier()

        @pl.when(s == 0)
        def _():
            pltpu.sync_copy(deg_sh, zbuf)
            pltpu.sync_copy(zbuf, out_hbm.at[pl.ds(c * n, n)])

    return deg


@functools.lru_cache()
def _scatter_kernel(n, e, f):
    epw = e // NW
    nchunks = epw // CHUNK
    rps = (n // (NS * 8)) * 8
    rem = n - NS * rps
    zrows = 16
    mesh = plsc.VectorSubcoreMesh(core_axis_name="c", subcore_axis_name="s",
                                  num_cores=NC, num_subcores=NS)

    @functools.partial(
        pl.kernel,
        out_type=jax.ShapeDtypeStruct((NC * n, f), jnp.float32),
        mesh=mesh,
        scratch_types=[
            pltpu.VMEM((epw,), jnp.int32),
            pltpu.VMEM((nchunks, CHUNK), jnp.int32),
            pltpu.VMEM((CHUNK, f), jnp.float32),
            pltpu.VMEM((CHUNK, f), jnp.float32),
            pltpu.VMEM((zrows, f), jnp.float32),
            pltpu.VMEM_SHARED((n, f), jnp.float32),
            pltpu.SemaphoreType.DMA,
            pltpu.SemaphoreType.DMA,
            pltpu.SemaphoreType.DMA,
            pltpu.SemaphoreType.DMA,
        ],
    )
    def scat(g_hbm, src_hbm, dst_hbm, out_hbm,
             sidx, didx, rows0, rows1, zbuf, acc_sh, semg0, semg1, sems0, sems1):
        c = lax.axis_index("c")
        s = lax.axis_index("s")
        w = c * NS + s

        pltpu.sync_copy(src_hbm.at[pl.ds(w * epw, epw)], sidx)
        pltpu.sync_copy(dst_hbm.at[w], didx)

        def zb(i, carry):
            for j in range(f // 16):
                zbuf[i, pl.ds(j * 16, 16)] = jnp.zeros((16,), jnp.float32)
            return carry

        lax.fori_loop(0, zrows, zb, 0)
        for t in range(rps // zrows):
            pltpu.sync_copy(zbuf, acc_sh.at[pl.ds(s * rps + t * zrows, zrows)])
        if rem:
            @pl.when(s == NS - 1)
            def _():
                pltpu.sync_copy(zbuf.at[pl.ds(0, rem)],
                                acc_sh.at[pl.ds(NS * rps, rem)])
        plsc.subcore_barrier()

        A = (rows0, semg0, sems0)
        B = (rows1, semg1, sems1)

        def gfire(k, buf):
            pltpu.async_copy(g_hbm.at[sidx.at[pl.ds(k * CHUNK, CHUNK)]],
                             buf[0], buf[1])

        def gwait(buf):
            pltpu.make_async_copy(g_hbm.at[sidx.at[pl.ds(0, CHUNK)]],
                                  buf[0], buf[1]).wait()

        def sfire(k, buf):
            pltpu.async_copy(buf[0], acc_sh.at[didx.at[k]], buf[2], add=True)

        def swait(buf):
            pltpu.make_async_copy(buf[0], acc_sh.at[didx.at[0]], buf[2]).wait()

        gfire(0, A)
        gwait(A)
        sfire(0, A)
        gfire(1, B)

        nloop = (nchunks - 3) // 2

        def body(j, carry):
            k = 2 * j + 1
            gwait(B)
            sfire(k, B)
            swait(A)
            gfire(k + 1, A)
            gwait(A)
            sfire(k + 1, A)
            swait(B)
            gfire(k + 2, B)
            return carry

        lax.fori_loop(0, nloop, body, 0)
        for k in range(2 * nloop + 1, nchunks):
            buf, other = (A, B) if k % 2 == 0 else (B, A)
            gwait(buf)
            sfire(k, buf)
            swait(other)
            if k + 1 < nchunks:
                gfire(k + 1, other)
        swait(A if (nchunks - 1) % 2 == 0 else B)
        plsc.subcore_barrier()

        pltpu.sync_copy(
            acc_sh.at[pl.ds(s * rps, rps)],
            out_hbm.at[pl.ds(c * n + s * rps, rps)],
        )
        if rem:
            @pl.when(s == NS - 1)
            def _():
                pltpu.sync_copy(acc_sh.at[pl.ds(NS * rps, rem)],
                                out_hbm.at[pl.ds(c * n + NS * rps, rem)])

    return scat


def _first_body(x_ref, w_ref, deg_ref, g_ref, dis_ref):
    d = lax.rsqrt(deg_ref[0] + deg_ref[1] + 1.0)
    dis_ref[...] = d
    g_ref[...] = jnp.dot(x_ref[...], w_ref[...],
                         preferred_element_type=jnp.float32) * d


@functools.lru_cache()
def _first_kernel(n, f_in, f_out):
    grid = n // BLK
    return pl.pallas_call(
        _first_body,
        grid=(grid,),
        in_specs=[
            pl.BlockSpec((BLK, f_in), lambda i: (i, 0)),
            pl.BlockSpec((f_in, f_out), lambda i: (0, 0)),
            pl.BlockSpec((NC, BLK, 1), lambda i: (0, i, 0)),
        ],
        out_specs=[
            pl.BlockSpec((BLK, f_out), lambda i: (i, 0)),
            pl.BlockSpec((BLK, 1), lambda i: (i, 0)),
        ],
        out_shape=[
            jax.ShapeDtypeStruct((n, f_out), jnp.float32),
            jax.ShapeDtypeStruct((n, 1), jnp.float32),
        ],
    )


def _mid_body(acc_ref, g_ref, dis_ref, b_ref, w_ref, o_ref):
    d = dis_ref[...]
    y = (acc_ref[0] + acc_ref[1] + g_ref[...]) * d + b_ref[...]
    y = jnp.maximum(y, 0.0)
    o_ref[...] = jnp.dot(y, w_ref[...],
                         preferred_element_type=jnp.float32) * d


@functools.lru_cache()
def _mid_kernel(n, f_in, f_out):
    grid = n // BLK
    return pl.pallas_call(
        _mid_body,
        grid=(grid,),
        in_specs=[
            pl.BlockSpec((NC, BLK, f_in), lambda i: (0, i, 0)),
            pl.BlockSpec((BLK, f_in), lambda i: (i, 0)),
            pl.BlockSpec((BLK, 1), lambda i: (i, 0)),
            pl.BlockSpec((f_in,), lambda i: (0,)),
            pl.BlockSpec((f_in, f_out), lambda i: (0, 0)),
        ],
        out_specs=pl.BlockSpec((BLK, f_out), lambda i: (i, 0)),
        out_shape=jax.ShapeDtypeStruct((n, f_out), jnp.float32),
    )


def _last_body(acc_ref, g_ref, dis_ref, b_ref, o_ref):
    d = dis_ref[...]
    o_ref[...] = (acc_ref[0] + acc_ref[1] + g_ref[...]) * d + b_ref[...]


@functools.lru_cache()
def _last_kernel(n, f):
    grid = n // BLK
    return pl.pallas_call(
        _last_body,
        grid=(grid,),
        in_specs=[
            pl.BlockSpec((NC, BLK, f), lambda i: (0, i, 0)),
            pl.BlockSpec((BLK, f), lambda i: (i, 0)),
            pl.BlockSpec((BLK, 1), lambda i: (i, 0)),
            pl.BlockSpec((f,), lambda i: (0,)),
        ],
        out_specs=pl.BlockSpec((BLK, f), lambda i: (i, 0)),
        out_shape=jax.ShapeDtypeStruct((n, f), jnp.float32),
    )


def kernel(x, edge_index, W1, b1, W2, b2, W3, b3, W4, b4):
    n, f_in = x.shape
    e = edge_index.shape[1]
    src = edge_index[0].astype(jnp.int32)
    dst = edge_index[1].astype(jnp.int32)

    if W2.shape[1] % 128:
        p = 128 - W2.shape[1] % 128
        W2 = jnp.pad(W2, ((0, 0), (0, p)))
        b2 = jnp.pad(b2, (0, p))
        W3 = jnp.pad(W3, ((0, p), (0, 0)))

    dst3 = dst.reshape(NW, (e // NW) // CHUNK, CHUNK)

    degs = _deg_kernel(n, e)(dst3).reshape(NC, n, 1)

    def agg(g):
        f = g.shape[1]
        acc = _scatter_kernel(n, e, f)(g, src, dst3)
        return acc.reshape(NC, n, f)

    g1, dis = _first_kernel(n, f_in, W1.shape[1])(x, W1, degs)
    g2 = _mid_kernel(n, W1.shape[1], W2.shape[1])(agg(g1), g1, dis, b1, W2)
    g3 = _mid_kernel(n, W2.shape[1], W3.shape[1])(agg(g2), g2, dis, b2, W3)
    g4 = _mid_kernel(n, W3.shape[1], W4.shape[1])(agg(g3), g3, dis, b3, W4)
    return _last_kernel(n, W4.shape[1])(agg(g4), g4, dis, b4)


# speedup vs baseline: 1.2600x; 1.2600x over previous
import functools

import jax
import jax.numpy as jnp
from jax import lax
from jax.experimental import pallas as pl
from jax.experimental.pallas import tpu as pltpu
from jax.experimental.pallas import tpu_sc as plsc

NC = 2
NS = 16
NW = NC * NS
CHUNK = 80
BLK = 2000


@functools.lru_cache()
def _deg_kernel(n, e):
    epw = e // NW
    nchunks = epw // CHUNK
    mesh = plsc.VectorSubcoreMesh(core_axis_name="c", subcore_axis_name="s",
                                  num_cores=NC, num_subcores=NS)

    depth = 8

    @functools.partial(
        pl.kernel,
        out_type=jax.ShapeDtypeStruct((NC * n,), jnp.float32),
        mesh=mesh,
        scratch_types=[
            pltpu.VMEM((CHUNK,), jnp.float32),
            pltpu.VMEM((nchunks, CHUNK), jnp.int32),
            pltpu.VMEM((n,), jnp.float32),
            pltpu.VMEM_SHARED((n,), jnp.float32),
            pltpu.SemaphoreType.DMA,
        ],
    )
    def deg(dst_hbm, out_hbm, ones_v, didx, zbuf, deg_sh, sem):
        c = lax.axis_index("c")
        s = lax.axis_index("s")
        w = c * NS + s

        for j in range(CHUNK // 16):
            ones_v[pl.ds(j * 16, 16)] = jnp.ones((16,), jnp.float32)
        pltpu.sync_copy(dst_hbm.at[w], didx)

        @pl.when(s == 0)
        def _():
            def zb(i, carry):
                zbuf[pl.ds(i * 16, 16)] = jnp.zeros((16,), jnp.float32)
                return carry

            lax.fori_loop(0, n // 16, zb, 0)
            pltpu.sync_copy(zbuf, deg_sh)

        plsc.subcore_barrier()

        def fire(k):
            pltpu.async_copy(ones_v, deg_sh.at[didx.at[k]], sem, add=True)

        def drain():
            pltpu.make_async_copy(ones_v, deg_sh.at[didx.at[0]], sem).wait()

        for k in range(depth):
            fire(k)

        def body(k, carry):
            drain()
            fire(k + depth)
            return carry

        lax.fori_loop(0, nchunks - depth, body, 0)
        for _ in range(depth):
            drain()
        plsc.subcore_barrier()

        @pl.when(s == 0)
        def _():
            pltpu.sync_copy(deg_sh, zbuf)
            pltpu.sync_copy(zbuf, out_hbm.at[pl.ds(c * n, n)])

    return deg


@functools.lru_cache()
def _scatter_kernel(n, e, f):
    epw = e // NW
    nchunks = epw // CHUNK
    rps = (n // (NS * 8)) * 8
    rem = n - NS * rps
    zrows = 16
    mesh = plsc.VectorSubcoreMesh(core_axis_name="c", subcore_axis_name="s",
                                  num_cores=NC, num_subcores=NS)

    @functools.partial(
        pl.kernel,
        out_type=jax.ShapeDtypeStruct((NC * n, f), jnp.float32),
        mesh=mesh,
        scratch_types=[
            pltpu.VMEM((epw,), jnp.int32),
            pltpu.VMEM((nchunks, CHUNK), jnp.int32),
            pltpu.VMEM((CHUNK, f), jnp.float32),
            pltpu.VMEM((CHUNK, f), jnp.float32),
            pltpu.VMEM((zrows, f), jnp.float32),
            pltpu.VMEM_SHARED((n, f), jnp.float32),
            pltpu.SemaphoreType.DMA,
            pltpu.SemaphoreType.DMA,
            pltpu.SemaphoreType.DMA,
            pltpu.SemaphoreType.DMA,
        ],
    )
    def scat(g_hbm, src_hbm, dst_hbm, out_hbm,
             sidx, didx, rows0, rows1, zbuf, acc_sh, semg0, semg1, sems0, sems1):
        c = lax.axis_index("c")
        s = lax.axis_index("s")
        w = c * NS + s

        pltpu.sync_copy(src_hbm.at[pl.ds(w * epw, epw)], sidx)
        pltpu.sync_copy(dst_hbm.at[w], didx)

        def zb(i, carry):
            for j in range(f // 16):
                zbuf[i, pl.ds(j * 16, 16)] = jnp.zeros((16,), jnp.float32)
            return carry

        lax.fori_loop(0, zrows, zb, 0)
        for t in range(rps // zrows):
            pltpu.sync_copy(zbuf, acc_sh.at[pl.ds(s * rps + t * zrows, zrows)])
        if rem:
            @pl.when(s == NS - 1)
            def _():
                pltpu.sync_copy(zbuf.at[pl.ds(0, rem)],
                                acc_sh.at[pl.ds(NS * rps, rem)])
        plsc.subcore_barrier()

        def gather(k, buf, sem):
            pltpu.async_copy(g_hbm.at[sidx.at[pl.ds(k * CHUNK, CHUNK)]], buf, sem)

        def gwait(buf, sem):
            pltpu.make_async_copy(g_hbm.at[sidx.at[pl.ds(0, CHUNK)]], buf, sem).wait()

        def scatter(k, buf):
            pltpu.sync_copy(buf, acc_sh.at[didx.at[k]], add=True)

        npairs = (nchunks - 1) // 2
        gather(0, rows0, semg0)

        def body(j, carry):
            k = 2 * j
            gather(k + 1, rows1, semg1)
            gwait(rows0, semg0)
            scatter(k, rows0)
            gather(k + 2, rows0, semg0)
            gwait(rows1, semg1)
            scatter(k + 1, rows1)
            return carry

        lax.fori_loop(0, npairs, body, 0)
        for k in range(2 * npairs, nchunks):
            buf, sem = (rows0, semg0) if k % 2 == 0 else (rows1, semg1)
            if k != 2 * npairs:
                gather(k, buf, sem)
            gwait(buf, sem)
            scatter(k, buf)
        plsc.subcore_barrier()

        pltpu.sync_copy(
            acc_sh.at[pl.ds(s * rps, rps)],
            out_hbm.at[pl.ds(c * n + s * rps, rps)],
        )
        if rem:
            @pl.when(s == NS - 1)
            def _():
                pltpu.sync_copy(acc_sh.at[pl.ds(NS * rps, rem)],
                                out_hbm.at[pl.ds(c * n + NS * rps, rem)])

    return scat


def _first_body(x_ref, w_ref, deg_ref, g_ref, dis_ref):
    d = lax.rsqrt(deg_ref[0] + deg_ref[1] + 1.0)
    dis_ref[...] = d
    g_ref[...] = jnp.dot(x_ref[...], w_ref[...],
                         preferred_element_type=jnp.float32) * d


@functools.lru_cache()
def _first_kernel(n, f_in, f_out):
    grid = n // BLK
    return pl.pallas_call(
        _first_body,
        grid=(grid,),
        in_specs=[
            pl.BlockSpec((BLK, f_in), lambda i: (i, 0)),
            pl.BlockSpec((f_in, f_out), lambda i: (0, 0)),
            pl.BlockSpec((NC, BLK, 1), lambda i: (0, i, 0)),
        ],
        out_specs=[
            pl.BlockSpec((BLK, f_out), lambda i: (i, 0)),
            pl.BlockSpec((BLK, 1), lambda i: (i, 0)),
        ],
        out_shape=[
            jax.ShapeDtypeStruct((n, f_out), jnp.float32),
            jax.ShapeDtypeStruct((n, 1), jnp.float32),
        ],
    )


def _mid_body(acc_ref, g_ref, dis_ref, b_ref, w_ref, o_ref):
    d = dis_ref[...]
    y = (acc_ref[0] + acc_ref[1] + g_ref[...]) * d + b_ref[...]
    y = jnp.maximum(y, 0.0)
    o_ref[...] = jnp.dot(y, w_ref[...],
                         preferred_element_type=jnp.float32) * d


@functools.lru_cache()
def _mid_kernel(n, f_in, f_out):
    grid = n // BLK
    return pl.pallas_call(
        _mid_body,
        grid=(grid,),
        in_specs=[
            pl.BlockSpec((NC, BLK, f_in), lambda i: (0, i, 0)),
            pl.BlockSpec((BLK, f_in), lambda i: (i, 0)),
            pl.BlockSpec((BLK, 1), lambda i: (i, 0)),
            pl.BlockSpec((f_in,), lambda i: (0,)),
            pl.BlockSpec((f_in, f_out), lambda i: (0, 0)),
        ],
        out_specs=pl.BlockSpec((BLK, f_out), lambda i: (i, 0)),
        out_shape=jax.ShapeDtypeStruct((n, f_out), jnp.float32),
    )


def _last_body(acc_ref, g_ref, dis_ref, b_ref, o_ref):
    d = dis_ref[...]
    o_ref[...] = (acc_ref[0] + acc_ref[1] + g_ref[...]) * d + b_ref[...]


@functools.lru_cache()
def _last_kernel(n, f):
    grid = n // BLK
    return pl.pallas_call(
        _last_body,
        grid=(grid,),
        in_specs=[
            pl.BlockSpec((NC, BLK, f), lambda i: (0, i, 0)),
            pl.BlockSpec((BLK, f), lambda i: (i, 0)),
            pl.BlockSpec((BLK, 1), lambda i: (i, 0)),
            pl.BlockSpec((f,), lambda i: (0,)),
        ],
        out_specs=pl.BlockSpec((BLK, f), lambda i: (i, 0)),
        out_shape=jax.ShapeDtypeStruct((n, f), jnp.float32),
    )


def kernel(x, edge_index, W1, b1, W2, b2, W3, b3, W4, b4):
    n, f_in = x.shape
    e = edge_index.shape[1]
    src = edge_index[0].astype(jnp.int32)
    dst = edge_index[1].astype(jnp.int32)

    if W2.shape[1] % 128:
        p = 128 - W2.shape[1] % 128
        W2 = jnp.pad(W2, ((0, 0), (0, p)))
        b2 = jnp.pad(b2, (0, p))
        W3 = jnp.pad(W3, ((0, p), (0, 0)))

    dst3 = dst.reshape(NW, (e // NW) // CHUNK, CHUNK)

    degs = _deg_kernel(n, e)(dst3).reshape(NC, n, 1)

    def agg(g):
        f = g.shape[1]
        acc = _scatter_kernel(n, e, f)(g, src, dst3)
        return acc.reshape(NC, n, f)

    g1, dis = _first_kernel(n, f_in, W1.shape[1])(x, W1, degs)
    g2 = _mid_kernel(n, W1.shape[1], W2.shape[1])(agg(g1), g1, dis, b1, W2)
    g3 = _mid_kernel(n, W2.shape[1], W3.shape[1])(agg(g2), g2, dis, b2, W3)
    g4 = _mid_kernel(n, W3.shape[1], W4.shape[1])(agg(g3), g3, dis, b3, W4)
    return _last_kernel(n, W4.shape[1])(agg(g4), g4, dis, b4)


# speedup vs baseline: 1.2798x; 1.0157x over previous
import functools

import jax
import jax.numpy as jnp
from jax import lax
from jax.experimental import pallas as pl
from jax.experimental.pallas import tpu as pltpu
from jax.experimental.pallas import tpu_sc as plsc

NC = 2
NS = 16
NW = NC * NS
CHUNK = 80
BLK = 2000


@functools.lru_cache()
def _deg_kernel(n, e):
    epw = e // NW
    nchunks = epw // CHUNK
    mesh = plsc.VectorSubcoreMesh(core_axis_name="c", subcore_axis_name="s",
                                  num_cores=NC, num_subcores=NS)

    depth = 8

    @functools.partial(
        pl.kernel,
        out_type=jax.ShapeDtypeStruct((NC * n,), jnp.float32),
        mesh=mesh,
        scratch_types=[
            pltpu.VMEM((CHUNK,), jnp.float32),
            pltpu.VMEM((nchunks, CHUNK), jnp.int32),
            pltpu.VMEM((n,), jnp.float32),
            pltpu.VMEM_SHARED((n,), jnp.float32),
            pltpu.SemaphoreType.DMA,
        ],
    )
    def deg(ei3_hbm, out_hbm, ones_v, didx, zbuf, deg_sh, sem):
        c = lax.axis_index("c")
        s = lax.axis_index("s")
        w = c * NS + s

        for j in range(CHUNK // 16):
            ones_v[pl.ds(j * 16, 16)] = jnp.ones((16,), jnp.float32)
        pltpu.sync_copy(ei3_hbm.at[NW + w], didx)

        @pl.when(s == 0)
        def _():
            def zb(i, carry):
                zbuf[pl.ds(i * 16, 16)] = jnp.zeros((16,), jnp.float32)
                return carry

            lax.fori_loop(0, n // 16, zb, 0)
            pltpu.sync_copy(zbuf, deg_sh)

        plsc.subcore_barrier()

        def fire(k):
            pltpu.async_copy(ones_v, deg_sh.at[didx.at[k]], sem, add=True)

        def drain():
            pltpu.make_async_copy(ones_v, deg_sh.at[didx.at[0]], sem).wait()

        for k in range(depth):
            fire(k)

        def body(k, carry):
            drain()
            fire(k + depth)
            return carry

        lax.fori_loop(0, nchunks - depth, body, 0)
        for _ in range(depth):
            drain()
        plsc.subcore_barrier()

        @pl.when(s == 0)
        def _():
            pltpu.sync_copy(deg_sh, zbuf)
            pltpu.sync_copy(zbuf, out_hbm.at[pl.ds(c * n, n)])

    return deg


@functools.lru_cache()
def _scatter_kernel(n, e, f):
    epw = e // NW
    nchunks = epw // CHUNK
    rps = (n // (NS * 8)) * 8
    rem = n - NS * rps
    zrows = 16
    mesh = plsc.VectorSubcoreMesh(core_axis_name="c", subcore_axis_name="s",
                                  num_cores=NC, num_subcores=NS)

    @functools.partial(
        pl.kernel,
        out_type=jax.ShapeDtypeStruct((NC * n, f), jnp.float32),
        mesh=mesh,
        scratch_types=[
            pltpu.VMEM((epw,), jnp.int32),
            pltpu.VMEM((nchunks, CHUNK), jnp.int32),
            pltpu.VMEM((CHUNK, f), jnp.float32),
            pltpu.VMEM((CHUNK, f), jnp.float32),
            pltpu.VMEM((zrows, f), jnp.float32),
            pltpu.VMEM_SHARED((n, f), jnp.float32),
            pltpu.SemaphoreType.DMA,
            pltpu.SemaphoreType.DMA,
            pltpu.SemaphoreType.DMA,
            pltpu.SemaphoreType.DMA,
        ],
    )
    def scat(g_hbm, ei_hbm, ei3_hbm, out_hbm,
             sidx, didx, rows0, rows1, zbuf, acc_sh, semg0, semg1, sems0, sems1):
        c = lax.axis_index("c")
        s = lax.axis_index("s")
        w = c * NS + s

        pltpu.sync_copy(ei_hbm.at[pl.ds(w * epw, epw)], sidx)
        pltpu.sync_copy(ei3_hbm.at[NW + w], didx)

        def zb(i, carry):
            for j in range(f // 16):
                zbuf[i, pl.ds(j * 16, 16)] = jnp.zeros((16,), jnp.float32)
            return carry

        lax.fori_loop(0, zrows, zb, 0)
        for t in range(rps // zrows):
            pltpu.sync_copy(zbuf, acc_sh.at[pl.ds(s * rps + t * zrows, zrows)])
        if rem:
            @pl.when(s == NS - 1)
            def _():
                pltpu.sync_copy(zbuf.at[pl.ds(0, rem)],
                                acc_sh.at[pl.ds(NS * rps, rem)])
        plsc.subcore_barrier()

        def gather(k, buf, sem):
            pltpu.async_copy(g_hbm.at[sidx.at[pl.ds(k * CHUNK, CHUNK)]], buf, sem)

        def gwait(buf, sem):
            pltpu.make_async_copy(g_hbm.at[sidx.at[pl.ds(0, CHUNK)]], buf, sem).wait()

        def scatter(k, buf):
            pltpu.sync_copy(buf, acc_sh.at[didx.at[k]], add=True)

        npairs = (nchunks - 1) // 2
        gather(0, rows0, semg0)

        def body(j, carry):
            k = 2 * j
            gather(k + 1, rows1, semg1)
            gwait(rows0, semg0)
            scatter(k, rows0)
            gather(k + 2, rows0, semg0)
            gwait(rows1, semg1)
            scatter(k + 1, rows1)
            return carry

        lax.fori_loop(0, npairs, body, 0)
        for k in range(2 * npairs, nchunks):
            buf, sem = (rows0, semg0) if k % 2 == 0 else (rows1, semg1)
            if k != 2 * npairs:
                gather(k, buf, sem)
            gwait(buf, sem)
            scatter(k, buf)
        plsc.subcore_barrier()

        pltpu.sync_copy(
            acc_sh.at[pl.ds(s * rps, rps)],
            out_hbm.at[pl.ds(c * n + s * rps, rps)],
        )
        if rem:
            @pl.when(s == NS - 1)
            def _():
                pltpu.sync_copy(acc_sh.at[pl.ds(NS * rps, rem)],
                                out_hbm.at[pl.ds(c * n + NS * rps, rem)])

    return scat


def _first_body(x_ref, w_ref, deg_ref, g_ref, dis_ref):
    d = lax.rsqrt(deg_ref[0] + deg_ref[1] + 1.0)
    dis_ref[...] = d
    g_ref[...] = jnp.dot(x_ref[...], w_ref[...],
                         preferred_element_type=jnp.float32) * d


@functools.lru_cache()
def _first_kernel(n, f_in, f_out):
    grid = n // BLK
    return pl.pallas_call(
        _first_body,
        grid=(grid,),
        in_specs=[
            pl.BlockSpec((BLK, f_in), lambda i: (i, 0)),
            pl.BlockSpec((f_in, f_out), lambda i: (0, 0)),
            pl.BlockSpec((NC, BLK, 1), lambda i: (0, i, 0)),
        ],
        out_specs=[
            pl.BlockSpec((BLK, f_out), lambda i: (i, 0)),
            pl.BlockSpec((BLK, 1), lambda i: (i, 0)),
        ],
        out_shape=[
            jax.ShapeDtypeStruct((n, f_out), jnp.float32),
            jax.ShapeDtypeStruct((n, 1), jnp.float32),
        ],
    )


def _mid_body(acc_ref, g_ref, dis_ref, b_ref, w_ref, o_ref):
    d = dis_ref[...]
    y = (acc_ref[0] + acc_ref[1] + g_ref[...]) * d + b_ref[...]
    y = jnp.maximum(y, 0.0)
    o_ref[...] = jnp.dot(y, w_ref[...],
                         preferred_element_type=jnp.float32) * d


@functools.lru_cache()
def _mid_kernel(n, f_in, f_out):
    grid = n // BLK
    return pl.pallas_call(
        _mid_body,
        grid=(grid,),
        in_specs=[
            pl.BlockSpec((NC, BLK, f_in), lambda i: (0, i, 0)),
            pl.BlockSpec((BLK, f_in), lambda i: (i, 0)),
            pl.BlockSpec((BLK, 1), lambda i: (i, 0)),
            pl.BlockSpec((f_in,), lambda i: (0,)),
            pl.BlockSpec((f_in, f_out), lambda i: (0, 0)),
        ],
        out_specs=pl.BlockSpec((BLK, f_out), lambda i: (i, 0)),
        out_shape=jax.ShapeDtypeStruct((n, f_out), jnp.float32),
    )


def _last_body(acc_ref, g_ref, dis_ref, b_ref, o_ref):
    d = dis_ref[...]
    o_ref[...] = (acc_ref[0] + acc_ref[1] + g_ref[...]) * d + b_ref[...]


@functools.lru_cache()
def _last_kernel(n, f):
    grid = n // BLK
    return pl.pallas_call(
        _last_body,
        grid=(grid,),
        in_specs=[
            pl.BlockSpec((NC, BLK, f), lambda i: (0, i, 0)),
            pl.BlockSpec((BLK, f), lambda i: (i, 0)),
            pl.BlockSpec((BLK, 1), lambda i: (i, 0)),
            pl.BlockSpec((f,), lambda i: (0,)),
        ],
        out_specs=pl.BlockSpec((BLK, f), lambda i: (i, 0)),
        out_shape=jax.ShapeDtypeStruct((n, f), jnp.float32),
    )


def kernel(x, edge_index, W1, b1, W2, b2, W3, b3, W4, b4):
    n, f_in = x.shape
    e = edge_index.shape[1]

    if W2.shape[1] % 128:
        p = 128 - W2.shape[1] % 128
        W2 = jnp.pad(W2, ((0, 0), (0, p)))
        b2 = jnp.pad(b2, (0, p))
        W3 = jnp.pad(W3, ((0, p), (0, 0)))

    nchunks = (e // NW) // CHUNK
    ei = edge_index.astype(jnp.int32).reshape(2 * e)
    ei3 = ei.reshape(2 * NW, nchunks, CHUNK)

    degs = _deg_kernel(n, e)(ei3).reshape(NC, n, 1)

    def agg(g):
        f = g.shape[1]
        acc = _scatter_kernel(n, e, f)(g, ei, ei3)
        return acc.reshape(NC, n, f)

    g1, dis = _first_kernel(n, f_in, W1.shape[1])(x, W1, degs)
    g2 = _mid_kernel(n, W1.shape[1], W2.shape[1])(agg(g1), g1, dis, b1, W2)
    g3 = _mid_kernel(n, W2.shape[1], W3.shape[1])(agg(g2), g2, dis, b2, W3)
    g4 = _mid_kernel(n, W3.shape[1], W4.shape[1])(agg(g3), g3, dis, b3, W4)
    return _last_kernel(n, W4.shape[1])(agg(g4), g4, dis, b4)
